# baseline (speedup 1.0000x reference)
import jax
import jax.numpy as jnp
import numpy as np
from jax.experimental import pallas as pl
from jax.experimental.pallas import tpu as pltpu

_PERM = [0, 2, 6, 3, 7, 1, 5, 8, 4]
_INV_PERM = list(np.argsort(np.array(_PERM)))
_L_EXPAND_NP = np.array([0, 1, 1, 1, 2, 2, 2, 2, 2])
_GATE_SEL = [None if _PERM[p] == 0 else (0 if _PERM[p] <= 3 else 1) for p in range(9)]

_EB = 256


def _smooth_leaky(x, alpha=0.2):
    return ((1 + alpha) / 2.0) * x + ((1 - alpha) / 2.0) * x * (2.0 * jax.nn.sigmoid(x) - 1.0)


def _edge_body(xs_ref, xd_ref, ef_ref, wig_ref, winv_ref,
               w0s_ref, w0d_ref, w0f_ref, b0_ref,
               w1s_ref, w1d_ref, w1f_ref,
               w2s_ref, w2d_ref, w2f_ref,
               lng_ref, lnb_ref, adot_ref,
               w02_ref, b02_ref, w12_ref, w22_ref,
               vrot_ref, eal_ref):
    XS = xs_ref[...]
    XD = xd_ref[...]
    EF = ef_ref[...]
    WG = wig_ref[...]
    WI = winv_ref[...]

    def rot_rows(S):
        out = []
        for p in range(9):
            acc = None
            for j in range(9):
                t = WG[:, _PERM[p] * 9 + j][:, None] * S[:, 32 * j:32 * j + 32]
                acc = t if acc is None else acc + t
            out.append(acc)
        return out

    es = rot_rows(XS)
    ed = rot_rows(XD)
    ef9 = rot_rows(EF)

    def cat(parts):
        return jnp.concatenate(parts, axis=1)

    def mm(a, b):
        return jax.lax.dot_general(a, b, dimension_numbers=(((1,), (0,)), ((), ())),
                                   preferred_element_type=jnp.float32)

    x0 = (mm(cat(es[0:3]), w0s_ref[...]) + mm(cat(ed[0:3]), w0d_ref[...])
          + mm(cat(ef9[0:3]), w0f_ref[...]) + b0_ref[...])
    y1a = (mm(cat(es[3:5]), w1s_ref[...]) + mm(cat(ed[3:5]), w1d_ref[...])
           + mm(cat(ef9[3:5]), w1f_ref[...]))
    y1b = (mm(cat(es[5:7]), w1s_ref[...]) + mm(cat(ed[5:7]), w1d_ref[...])
           + mm(cat(ef9[5:7]), w1f_ref[...]))
    m1r = y1a[:, :64] - y1b[:, 64:]
    m1i = y1b[:, :64] + y1a[:, 64:]
    y2a = mm(es[7], w2s_ref[...]) + mm(ed[7], w2d_ref[...]) + mm(ef9[7], w2f_ref[...])
    y2b = mm(es[8], w2s_ref[...]) + mm(ed[8], w2d_ref[...]) + mm(ef9[8], w2f_ref[...])
    m2r = y2a[:, :32] - y2b[:, 32:]
    m2i = y2b[:, :32] + y2a[:, 32:]

    x0_alpha = x0[:, 0:64]
    g0 = jax.nn.sigmoid(x0[:, 64:96])
    g1 = jax.nn.sigmoid(x0[:, 96:128])
    x0m = x0[:, 128:224]

    rows1 = [x0m[:, 0:32], x0m[:, 32:64], x0m[:, 64:96],
             m1r[:, 0:32], m1r[:, 32:64], m1i[:, 0:32], m1i[:, 32:64],
             m2r, m2i]
    z = []
    for p in range(9):
        if _GATE_SEL[p] is None:
            z.append(rows1[p] * jax.nn.sigmoid(rows1[p]))
        elif _GATE_SEL[p] == 0:
            z.append(rows1[p] * g0)
        else:
            z.append(rows1[p] * g1)

    x02 = mm(cat(z[0:3]), w02_ref[...]) + b02_ref[...]
    y1a2 = mm(cat(z[3:5]), w12_ref[...])
    y1b2 = mm(cat(z[5:7]), w12_ref[...])
    m1r2 = y1a2[:, :64] - y1b2[:, 64:]
    m1i2 = y1b2[:, :64] + y1a2[:, 64:]
    y2a2 = mm(z[7], w22_ref[...])
    y2b2 = mm(z[8], w22_ref[...])
    m2r2 = y2a2[:, :32] - y2b2[:, 32:]
    m2i2 = y2b2[:, :32] + y2a2[:, 32:]
    Z2 = [x02[:, 0:32], x02[:, 32:64], x02[:, 64:96],
          m1r2[:, 0:32], m1r2[:, 32:64], m1i2[:, 0:32], m1i2[:, 32:64],
          m2r2, m2i2]

    lng = lng_ref[...]
    lnb = lnb_ref[...]
    eal_cols = []
    for h in range(4):
        ah = x0_alpha[:, 16 * h:16 * h + 16]
        mu = jnp.mean(ah, axis=1, keepdims=True)
        d = ah - mu
        var = jnp.mean(d * d, axis=1, keepdims=True)
        a = d * jax.lax.rsqrt(var + 1e-5) * lng + lnb
        a = _smooth_leaky(a)
        al = jnp.sum(a * adot_ref[h, :][None, :], axis=1, keepdims=True)
        eal_cols.append(jnp.exp(al))
    eal_ref[...] = jnp.concatenate(eal_cols, axis=1)

    vcols = []
    for i in range(9):
        acc = None
        for r in range(9):
            t = WI[:, i * 9 + r][:, None] * Z2[_INV_PERM[r]]
            acc = t if acc is None else acc + t
        vcols.append(acc)
    vrot_ref[...] = jnp.concatenate(vcols, axis=1)


def _edge_compute(xs, xd, ef, wig, winv, weights):
    (w0s, w0d, w0f, b0, w1s, w1d, w1f, w2s, w2d, w2f,
     lng, lnb, adot, w02, b02, w12, w22) = weights
    E = xs.shape[0]
    B = _EB
    grid = (E // B,)
    espec = lambda cols: pl.BlockSpec((B, cols), lambda e: (e, 0))
    wspec = lambda r, c: pl.BlockSpec((r, c), lambda e: (0, 0))
    vrot, eal = pl.pallas_call(
        _edge_body,
        grid=grid,
        in_specs=[espec(288), espec(288), espec(288), espec(81), espec(81),
                  wspec(96, 224), wspec(96, 224), wspec(96, 224), wspec(1, 224),
                  wspec(64, 128), wspec(64, 128), wspec(64, 128),
                  wspec(32, 64), wspec(32, 64), wspec(32, 64),
                  wspec(1, 16), wspec(1, 16), wspec(4, 16),
                  wspec(96, 96), wspec(1, 96), wspec(64, 128), wspec(32, 64)],
        out_specs=[pl.BlockSpec((B, 288), lambda e: (e, 0)),
                   pl.BlockSpec((B, 4), lambda e: (e, 0))],
        out_shape=[jax.ShapeDtypeStruct((E, 288), jnp.float32),
                   jax.ShapeDtypeStruct((E, 4), jnp.float32)],
    )(xs, xd, ef, wig, winv, w0s, w0d, w0f, b0, w1s, w1d, w1f,
      w2s, w2d, w2f, lng, lnb, adot, w02, b02, w12, w22)
    return vrot, eal


def _split3(w, ncoeff, cout):
    w3 = w.reshape(ncoeff, 3, 32, cout)
    return (w3[:, 0].reshape(ncoeff * 32, cout),
            w3[:, 1].reshape(ncoeff * 32, cout),
            w3[:, 2].reshape(ncoeff * 32, cout))


def _proj_body(node_ref, pw_ref, pb_ref, out_ref):
    m = pl.program_id(0)
    acc = jax.lax.dot_general(
        node_ref[0], pw_ref[0],
        dimension_numbers=(((1,), (1,)), ((), ())),
        preferred_element_type=jnp.float32)

    @pl.when(m == 0)
    def _():
        out_ref[0] = acc + pb_ref[...][0][None, :]

    @pl.when(m != 0)
    def _():
        out_ref[0] = acc


def _so3_project(node, proj_w, proj_b):
    N, M, C = node.shape
    OUT = proj_w.shape[1]
    node_t = jnp.transpose(node, (1, 0, 2))
    pw = proj_w[_L_EXPAND_NP]
    out_t = pl.pallas_call(
        _proj_body,
        grid=(M,),
        in_specs=[
            pl.BlockSpec((1, N, C), lambda m: (m, 0, 0)),
            pl.BlockSpec((1, OUT, C), lambda m: (m, 0, 0)),
            pl.BlockSpec((1, OUT), lambda m: (0, 0)),
        ],
        out_specs=pl.BlockSpec((1, N, OUT), lambda m: (m, 0, 0)),
        out_shape=jax.ShapeDtypeStruct((M, N, OUT), jnp.float32),
    )(node_t, pw, proj_b.reshape(1, OUT))
    return jnp.transpose(out_t, (1, 0, 2))


def kernel(x, atomic_numbers, edge_distance, edge_index, edge_fea, wigner, wigner_inv,
           src_emb, tgt_emb, w0_1, b0_1, w1_1, w2_1, ln_gamma, ln_beta, alpha_dot,
           w0_2, b0_2, w1_2, w2_2, proj_w, proj_b):
    N = x.shape[0]
    E = edge_index.shape[1]
    src, dst = edge_index[0], edge_index[1]

    x2 = x.reshape(N, 288)
    xs = x2[src]
    xd = x2[dst]
    ef = edge_fea.reshape(E, 288)
    wig = wigner.reshape(E, 81)
    winv = wigner_inv.reshape(E, 81)

    w0s, w0d, w0f = _split3(w0_1, 3, 224)
    w1s, w1d, w1f = _split3(w1_1, 2, 128)
    w2s, w2d, w2f = _split3(w2_1, 1, 64)
    weights = (w0s, w0d, w0f, b0_1.reshape(1, 224),
               w1s, w1d, w1f, w2s, w2d, w2f,
               ln_gamma.reshape(1, 16), ln_beta.reshape(1, 16), alpha_dot,
               w0_2, b0_2.reshape(1, 96), w1_2, w2_2)

    vrot, eal = _edge_compute(xs, xd, ef, wig, winv, weights)

    den = jax.ops.segment_sum(eal, dst, num_segments=N)
    alnorm = eal / den[dst]
    attn = vrot.reshape(E, 9, 4, 8) * alnorm[:, None, :, None]
    node = jax.ops.segment_sum(attn.reshape(E, 9, 32), dst, num_segments=N)
    return _so3_project(node, proj_w, proj_b)


# speedup vs baseline: 1.0162x; 1.0162x over previous
import functools

import jax
import jax.numpy as jnp
import numpy as np
from jax import lax
from jax.experimental import pallas as pl
from jax.experimental.pallas import tpu as pltpu
from jax.experimental.pallas import tpu_sc as plsc

_PERM = [0, 2, 6, 3, 7, 1, 5, 8, 4]
_INV_PERM = list(np.argsort(np.array(_PERM)))
_L_EXPAND_NP = np.array([0, 1, 1, 1, 2, 2, 2, 2, 2])
_GATE_SEL = [None if _PERM[p] == 0 else (0 if _PERM[p] <= 3 else 1) for p in range(9)]

_EB = 256


def _smooth_leaky(x, alpha=0.2):
    return ((1 + alpha) / 2.0) * x + ((1 - alpha) / 2.0) * x * (2.0 * jax.nn.sigmoid(x) - 1.0)


def _edge_body(xs_ref, xd_ref, ef_ref, wig_ref, winv_ref,
               w0s_ref, w0d_ref, w0f_ref, b0_ref,
               w1s_ref, w1d_ref, w1f_ref,
               w2s_ref, w2d_ref, w2f_ref,
               lng_ref, lnb_ref, adot_ref,
               w02_ref, b02_ref, w12_ref, w22_ref,
               vrot_ref, eal_ref):
    XS = xs_ref[...]
    XD = xd_ref[...]
    EF = ef_ref[...]
    WG = wig_ref[...]
    WI = winv_ref[...]

    def rot_rows(S):
        out = []
        for p in range(9):
            acc = None
            for j in range(9):
                t = WG[:, _PERM[p] * 9 + j][:, None] * S[:, 32 * j:32 * j + 32]
                acc = t if acc is None else acc + t
            out.append(acc)
        return out

    es = rot_rows(XS)
    ed = rot_rows(XD)
    ef9 = rot_rows(EF)

    def cat(parts):
        return jnp.concatenate(parts, axis=1)

    def mm(a, b):
        return jax.lax.dot_general(a, b, dimension_numbers=(((1,), (0,)), ((), ())),
                                   preferred_element_type=jnp.float32)

    x0 = (mm(cat(es[0:3]), w0s_ref[...]) + mm(cat(ed[0:3]), w0d_ref[...])
          + mm(cat(ef9[0:3]), w0f_ref[...]) + b0_ref[...])
    y1a = (mm(cat(es[3:5]), w1s_ref[...]) + mm(cat(ed[3:5]), w1d_ref[...])
           + mm(cat(ef9[3:5]), w1f_ref[...]))
    y1b = (mm(cat(es[5:7]), w1s_ref[...]) + mm(cat(ed[5:7]), w1d_ref[...])
           + mm(cat(ef9[5:7]), w1f_ref[...]))
    m1r = y1a[:, :64] - y1b[:, 64:]
    m1i = y1b[:, :64] + y1a[:, 64:]
    y2a = mm(es[7], w2s_ref[...]) + mm(ed[7], w2d_ref[...]) + mm(ef9[7], w2f_ref[...])
    y2b = mm(es[8], w2s_ref[...]) + mm(ed[8], w2d_ref[...]) + mm(ef9[8], w2f_ref[...])
    m2r = y2a[:, :32] - y2b[:, 32:]
    m2i = y2b[:, :32] + y2a[:, 32:]

    x0_alpha = x0[:, 0:64]
    g0 = jax.nn.sigmoid(x0[:, 64:96])
    g1 = jax.nn.sigmoid(x0[:, 96:128])
    x0m = x0[:, 128:224]

    rows1 = [x0m[:, 0:32], x0m[:, 32:64], x0m[:, 64:96],
             m1r[:, 0:32], m1r[:, 32:64], m1i[:, 0:32], m1i[:, 32:64],
             m2r, m2i]
    z = []
    for p in range(9):
        if _GATE_SEL[p] is None:
            z.append(rows1[p] * jax.nn.sigmoid(rows1[p]))
        elif _GATE_SEL[p] == 0:
            z.append(rows1[p] * g0)
        else:
            z.append(rows1[p] * g1)

    x02 = mm(cat(z[0:3]), w02_ref[...]) + b02_ref[...]
    y1a2 = mm(cat(z[3:5]), w12_ref[...])
    y1b2 = mm(cat(z[5:7]), w12_ref[...])
    m1r2 = y1a2[:, :64] - y1b2[:, 64:]
    m1i2 = y1b2[:, :64] + y1a2[:, 64:]
    y2a2 = mm(z[7], w22_ref[...])
    y2b2 = mm(z[8], w22_ref[...])
    m2r2 = y2a2[:, :32] - y2b2[:, 32:]
    m2i2 = y2b2[:, :32] + y2a2[:, 32:]
    Z2 = [x02[:, 0:32], x02[:, 32:64], x02[:, 64:96],
          m1r2[:, 0:32], m1r2[:, 32:64], m1i2[:, 0:32], m1i2[:, 32:64],
          m2r2, m2i2]

    lng = lng_ref[...]
    lnb = lnb_ref[...]
    eal_cols = []
    for h in range(4):
        ah = x0_alpha[:, 16 * h:16 * h + 16]
        mu = jnp.mean(ah, axis=1, keepdims=True)
        d = ah - mu
        var = jnp.mean(d * d, axis=1, keepdims=True)
        a = d * jax.lax.rsqrt(var + 1e-5) * lng + lnb
        a = _smooth_leaky(a)
        al = jnp.sum(a * adot_ref[h, :][None, :], axis=1, keepdims=True)
        eal_cols.append(jnp.exp(al))
    eal_ref[...] = jnp.concatenate(eal_cols, axis=1)

    vcols = []
    for i in range(9):
        acc = None
        for r in range(9):
            t = WI[:, i * 9 + r][:, None] * Z2[_INV_PERM[r]]
            acc = t if acc is None else acc + t
        vcols.append(acc)
    vrot_ref[...] = jnp.concatenate(vcols, axis=1)


def _edge_compute(xs, xd, ef, wig, winv, weights):
    (w0s, w0d, w0f, b0, w1s, w1d, w1f, w2s, w2d, w2f,
     lng, lnb, adot, w02, b02, w12, w22) = weights
    E = xs.shape[0]
    B = _EB
    grid = (E // B,)
    espec = lambda cols: pl.BlockSpec((B, cols), lambda e: (e, 0))
    wspec = lambda r, c: pl.BlockSpec((r, c), lambda e: (0, 0))
    vrot, eal = pl.pallas_call(
        _edge_body,
        grid=grid,
        in_specs=[espec(xs.shape[1]), espec(xd.shape[1]), espec(288), espec(81), espec(81),
                  wspec(96, 224), wspec(96, 224), wspec(96, 224), wspec(1, 224),
                  wspec(64, 128), wspec(64, 128), wspec(64, 128),
                  wspec(32, 64), wspec(32, 64), wspec(32, 64),
                  wspec(1, 16), wspec(1, 16), wspec(4, 16),
                  wspec(96, 96), wspec(1, 96), wspec(64, 128), wspec(32, 64)],
        out_specs=[pl.BlockSpec((B, 288), lambda e: (e, 0)),
                   pl.BlockSpec((B, 4), lambda e: (e, 0))],
        out_shape=[jax.ShapeDtypeStruct((E, 288), jnp.float32),
                   jax.ShapeDtypeStruct((E, 4), jnp.float32)],
    )(xs, xd, ef, wig, winv, w0s, w0d, w0f, b0, w1s, w1d, w1f,
      w2s, w2d, w2f, lng, lnb, adot, w02, b02, w12, w22)
    return vrot, eal


_NW = 32
_GK = 200


def _sc_gather(x2, src, dst):
    E = src.shape[0]
    D = x2.shape[1]
    per_w = E // _NW
    nchunk = per_w // _GK
    mesh = plsc.VectorSubcoreMesh(core_axis_name="c", subcore_axis_name="s")

    @functools.partial(
        pl.kernel, mesh=mesh,
        out_type=[jax.ShapeDtypeStruct((E, D), jnp.float32),
                  jax.ShapeDtypeStruct((E, D), jnp.float32)],
        scratch_types=[pltpu.VMEM((_GK,), jnp.int32),
                       pltpu.VMEM((_GK, D), jnp.float32),
                       pltpu.SemaphoreType.DMA],
    )
    def gk(x_hbm, src_hbm, dst_hbm, xs_hbm, xd_hbm, idx_v, rows_v, sem):
        wid = lax.axis_index("s") * 2 + lax.axis_index("c")
        base = wid * per_w

        def body(k, carry):
            off = base + k * _GK
            pltpu.sync_copy(src_hbm.at[pl.ds(off, _GK)], idx_v)
            pltpu.async_copy(x_hbm.at[idx_v], rows_v, sem).wait()
            pltpu.sync_copy(rows_v, xs_hbm.at[pl.ds(off, _GK)])
            pltpu.sync_copy(dst_hbm.at[pl.ds(off, _GK)], idx_v)
            pltpu.async_copy(x_hbm.at[idx_v], rows_v, sem).wait()
            pltpu.sync_copy(rows_v, xd_hbm.at[pl.ds(off, _GK)])
            return carry

        lax.fori_loop(0, nchunk, body, 0)

    return gk(x2, src, dst)


def _split3(w, ncoeff, cout):
    w3 = w.reshape(ncoeff, 3, 32, cout)
    return (w3[:, 0].reshape(ncoeff * 32, cout),
            w3[:, 1].reshape(ncoeff * 32, cout),
            w3[:, 2].reshape(ncoeff * 32, cout))


def _proj_body(node_ref, pw_ref, pb_ref, out_ref):
    m = pl.program_id(0)
    acc = jax.lax.dot_general(
        node_ref[0], pw_ref[0],
        dimension_numbers=(((1,), (1,)), ((), ())),
        preferred_element_type=jnp.float32)

    @pl.when(m == 0)
    def _():
        out_ref[0] = acc + pb_ref[...][0][None, :]

    @pl.when(m != 0)
    def _():
        out_ref[0] = acc


def _so3_project(node, proj_w, proj_b):
    N, M, C = node.shape
    OUT = proj_w.shape[1]
    node_t = jnp.transpose(node, (1, 0, 2))
    pw = proj_w[_L_EXPAND_NP]
    out_t = pl.pallas_call(
        _proj_body,
        grid=(M,),
        in_specs=[
            pl.BlockSpec((1, N, C), lambda m: (m, 0, 0)),
            pl.BlockSpec((1, OUT, C), lambda m: (m, 0, 0)),
            pl.BlockSpec((1, OUT), lambda m: (0, 0)),
        ],
        out_specs=pl.BlockSpec((1, N, OUT), lambda m: (m, 0, 0)),
        out_shape=jax.ShapeDtypeStruct((M, N, OUT), jnp.float32),
    )(node_t, pw, proj_b.reshape(1, OUT))
    return jnp.transpose(out_t, (1, 0, 2))


def kernel(x, atomic_numbers, edge_distance, edge_index, edge_fea, wigner, wigner_inv,
           src_emb, tgt_emb, w0_1, b0_1, w1_1, w2_1, ln_gamma, ln_beta, alpha_dot,
           w0_2, b0_2, w1_2, w2_2, proj_w, proj_b):
    N = x.shape[0]
    E = edge_index.shape[1]
    src, dst = edge_index[0], edge_index[1]

    x2 = jnp.pad(x.reshape(N, 288), ((0, 0), (0, 96)))
    xs, xd = _sc_gather(x2, src, dst)
    ef = edge_fea.reshape(E, 288)
    wig = wigner.reshape(E, 81)
    winv = wigner_inv.reshape(E, 81)

    w0s, w0d, w0f = _split3(w0_1, 3, 224)
    w1s, w1d, w1f = _split3(w1_1, 2, 128)
    w2s, w2d, w2f = _split3(w2_1, 1, 64)
    weights = (w0s, w0d, w0f, b0_1.reshape(1, 224),
               w1s, w1d, w1f, w2s, w2d, w2f,
               ln_gamma.reshape(1, 16), ln_beta.reshape(1, 16), alpha_dot,
               w0_2, b0_2.reshape(1, 96), w1_2, w2_2)

    vrot, eal = _edge_compute(xs, xd, ef, wig, winv, weights)

    den = jax.ops.segment_sum(eal, dst, num_segments=N)
    alnorm = eal / den[dst]
    attn = vrot.reshape(E, 9, 4, 8) * alnorm[:, None, :, None]
    node = jax.ops.segment_sum(attn.reshape(E, 9, 32), dst, num_segments=N)
    return _so3_project(node, proj_w, proj_b)
